# baseline (speedup 1.0000x reference)
import functools

import jax
import jax.numpy as jnp
from jax import lax
from jax.experimental import pallas as pl
from jax.experimental.pallas import tpu as pltpu
from jax.experimental.pallas import tpu_sc as plsc

_B = 16384
_D = 128
_NTABLE = 64
_NC = 2
_NS = 16
_L = 16
_NW = _NC * _NS
_BPW = _B // _NW
_CHUNK = 64
_NCHUNK = _BPW // _CHUNK


def _grid_index(w, inv_step):
    c = w * inv_step - 0.5
    t = c.astype(jnp.int32)
    up = jnp.where(c > t.astype(jnp.float32), 1, 0)
    return jnp.minimum(jnp.maximum(t + up, 0), _NTABLE // 8 - 1)


@functools.cache
def _build():
    @functools.partial(
        pl.kernel,
        out_type=jax.ShapeDtypeStruct((_B, _D), jnp.float32),
        mesh=plsc.VectorSubcoreMesh(core_axis_name="c", subcore_axis_name="s"),
        scratch_types=[
            pltpu.VMEM((_BPW,), jnp.float32),
            pltpu.VMEM((_BPW,), jnp.float32),
            pltpu.VMEM((_NCHUNK, _CHUNK), jnp.int32),
            pltpu.VMEM((_BPW, _D), jnp.float32),
            pltpu.VMEM_SHARED((_NTABLE, _D), jnp.float32),
            pltpu.SemaphoreType.DMA,
            pltpu.SemaphoreType.DMA,
            pltpu.SemaphoreType.DMA,
        ],
    )
    def _sc_lookup(
        cw_hbm, xw_hbm, table_hbm, out_hbm, cw_v, xw_v, idx_v, rows_v, table_sh, sem, out_sem, hsem
    ):
        cid = lax.axis_index("c")
        sid = lax.axis_index("s")
        wid = sid * _NC + cid
        base = wid * _BPW
        @pl.when(sid == 0)
        def _stage_table():
            pltpu.async_copy(table_hbm, table_sh, out_sem)

        in_cw = pltpu.async_copy(cw_hbm.at[pl.ds(base, _BPW)], cw_v, sem)
        in_xw = pltpu.async_copy(xw_hbm.at[pl.ds(base, _BPW)], xw_v, sem)
        in_cw.wait()
        in_xw.wait()

        vpc = _CHUNK // _L

        def _compute(i):
            ci = _grid_index(cw_v[pl.ds(i * _L, _L)], 1.0)
            xi = _grid_index(xw_v[pl.ds(i * _L, _L)], 2.0)
            idx_v[i // vpc, pl.ds((i % vpc) * _L, _L)] = ci + xi * 8

        last = _NCHUNK - 1
        for i in range(last * vpc, _BPW // _L):
            _compute(i)
        hbm_cp = pltpu.async_copy(
            table_hbm.at[idx_v.at[last]],
            rows_v.at[pl.ds(last * _CHUNK, _CHUNK)],
            hsem,
        )
        for i in range(last * vpc):
            _compute(i)

        @pl.when(sid == 0)
        def _wait_table():
            pltpu.make_async_copy(table_hbm, table_sh, out_sem).wait()

        plsc.subcore_barrier()

        copies = [
            pltpu.async_copy(
                table_sh.at[idx_v.at[j]],
                rows_v.at[pl.ds(j * _CHUNK, _CHUNK)],
                sem,
            )
            for j in range(last)
        ]
        out_copies = []
        for j in range(last):
            copies[j].wait()
            out_copies.append(
                pltpu.async_copy(
                    rows_v.at[pl.ds(j * _CHUNK, _CHUNK)],
                    out_hbm.at[pl.ds(base + j * _CHUNK, _CHUNK)],
                    out_sem,
                )
            )
        hbm_cp.wait()
        out_copies.append(
            pltpu.async_copy(
                rows_v.at[pl.ds(last * _CHUNK, _CHUNK)],
                out_hbm.at[pl.ds(base + last * _CHUNK, _CHUNK)],
                out_sem,
            )
        )
        for oc in out_copies:
            oc.wait()

    return _sc_lookup


def kernel(class_ws, x_cond_ws, embedding_table):
    return _build()(
        class_ws.astype(jnp.float32),
        x_cond_ws.astype(jnp.float32),
        embedding_table.astype(jnp.float32),
    )


# speedup vs baseline: 1.1530x; 1.1530x over previous
import functools

import jax
import jax.numpy as jnp
from jax import lax
from jax.experimental import pallas as pl
from jax.experimental.pallas import tpu as pltpu
from jax.experimental.pallas import tpu_sc as plsc

_B = 16384
_D = 128
_NTABLE = 64
_NC = 2
_NS = 16
_L = 16
_NW = _NC * _NS
_BPW = _B // _NW
_CHUNK = 64
_NCHUNK = _BPW // _CHUNK


def _grid_index(w, inv_step):
    c = w * inv_step - 0.5
    t = c.astype(jnp.int32)
    up = jnp.where(c > t.astype(jnp.float32), 1, 0)
    return jnp.minimum(jnp.maximum(t + up, 0), _NTABLE // 8 - 1)


@functools.cache
def _build():
    @functools.partial(
        pl.kernel,
        out_type=jax.ShapeDtypeStruct((_B, _D), jnp.float32),
        mesh=plsc.VectorSubcoreMesh(core_axis_name="c", subcore_axis_name="s"),
        scratch_types=[
            pltpu.VMEM((_BPW,), jnp.float32),
            pltpu.VMEM((_BPW,), jnp.float32),
            pltpu.VMEM((_NCHUNK, _CHUNK), jnp.int32),
            pltpu.VMEM((_BPW, _D), jnp.float32),
            pltpu.VMEM_SHARED((_NTABLE, _D), jnp.float32),
            pltpu.SemaphoreType.DMA,
            pltpu.SemaphoreType.DMA,
        ],
    )
    def _sc_lookup(
        cw_hbm, xw_hbm, table_hbm, out_hbm, cw_v, xw_v, idx_v, rows_v, table_sh, sem, out_sem
    ):
        cid = lax.axis_index("c")
        sid = lax.axis_index("s")
        wid = sid * _NC + cid
        base = wid * _BPW
        @pl.when(sid == 0)
        def _stage_table():
            pltpu.async_copy(table_hbm, table_sh, out_sem)

        in_cw = pltpu.async_copy(cw_hbm.at[pl.ds(base, _BPW)], cw_v, sem)
        in_xw = pltpu.async_copy(xw_hbm.at[pl.ds(base, _BPW)], xw_v, sem)
        in_cw.wait()
        in_xw.wait()

        for i in range(_BPW // _L):
            ci = _grid_index(cw_v[pl.ds(i * _L, _L)], 1.0)
            xi = _grid_index(xw_v[pl.ds(i * _L, _L)], 2.0)
            idx_v[i // (_CHUNK // _L), pl.ds((i % (_CHUNK // _L)) * _L, _L)] = ci + xi * 8

        @pl.when(sid == 0)
        def _wait_table():
            pltpu.make_async_copy(table_hbm, table_sh, out_sem).wait()

        plsc.subcore_barrier()

        copies = [
            pltpu.async_copy(
                table_sh.at[idx_v.at[j]],
                rows_v.at[pl.ds(j * _CHUNK, _CHUNK)],
                sem,
            )
            for j in range(_NCHUNK)
        ]
        out_copies = []
        for j in range(_NCHUNK):
            copies[j].wait()
            out_copies.append(
                pltpu.async_copy(
                    rows_v.at[pl.ds(j * _CHUNK, _CHUNK)],
                    out_hbm.at[pl.ds(base + j * _CHUNK, _CHUNK)],
                    out_sem,
                )
            )
        for oc in out_copies:
            oc.wait()

    return _sc_lookup


def kernel(class_ws, x_cond_ws, embedding_table):
    return _build()(
        class_ws.astype(jnp.float32),
        x_cond_ws.astype(jnp.float32),
        embedding_table.astype(jnp.float32),
    )
